# Initial kernel scaffold; baseline (speedup 1.0000x reference)
import jax
import jax.numpy as jnp
from jax.experimental import pallas as pl


def kernel(x, edge_idx, edge_attr, We, be, W1, b1, g1, bt1, W2, b2, g2, bt2, gn, btn, eps):
    raise NotImplementedError("write your pallas kernel here")


# speedup vs baseline: 2.5411x; 2.5411x over previous
import functools

import jax
import jax.numpy as jnp
from jax import lax
from jax.experimental import pallas as pl
from jax.experimental.pallas import tpu as pltpu
from jax.experimental.pallas import tpu_sc as plsc

N = 10000
E = 320000
F = 128
H = 256
BN_EPS = 1e-5
ALPHA = 0.01

NC = 2
NS = 16
NW = NC * NS
EPW = E // NW
CH = 80
NCHUNK = EPW // CH
ZB = 200
NZC = N // ZB
NF = F // 16
GARB = N


def _rne_bf16(v):
    c = v * jnp.float32(65537.0)
    return c - (c - v)


def _sc_body(x_hbm, src_hbm, dst_hbm, a0_hbm, a1_hbm, wb_hbm, out_hbm,
             srcv, dstv, a0v, a1v, rows, wbv, zbuf, idx16, agg, sem):
    cid = lax.axis_index("c")
    sid = lax.axis_index("s")
    wid = sid * NC + cid

    zv = jnp.zeros((16,), jnp.float32)

    def zrow(i, carry):
        for f in range(NF):
            zbuf[i, pl.ds(f * 16, 16)] = zv
        return carry

    lax.fori_loop(0, ZB, zrow, 0)

    def zchunk(k, carry):
        c = sid + k * NS

        @pl.when(c < NZC)
        def _():
            pltpu.sync_copy(zbuf, agg.at[pl.ds(c * ZB, ZB)])

        return carry

    lax.fori_loop(0, (NZC + NS - 1) // NS, zchunk, 0)

    @pl.when(sid == 0)
    def _():
        pltpu.sync_copy(zbuf.at[pl.ds(0, 8)], agg.at[pl.ds(GARB, 8)])

    pltpu.sync_copy(wb_hbm, wbv)
    w0 = [_rne_bf16(wbv[0, pl.ds(f * 16, 16)]) for f in range(NF)]
    w1 = [_rne_bf16(wbv[1, pl.ds(f * 16, 16)]) for f in range(NF)]
    bb = [wbv[2, pl.ds(f * 16, 16)] for f in range(NF)]

    plsc.subcore_barrier()

    ebase = wid * EPW

    def chunk_body(c, carry):
        off = ebase + c * CH
        pltpu.sync_copy(src_hbm.at[pl.ds(off, CH)], srcv)
        pltpu.sync_copy(dst_hbm.at[pl.ds(off, CH)], dstv)
        pltpu.sync_copy(a0_hbm.at[pl.ds(off, CH)], a0v)
        pltpu.sync_copy(a1_hbm.at[pl.ds(off, CH)], a1v)
        pltpu.async_copy(x_hbm.at[srcv], rows, sem).wait()

        def group_body(g, gcarry):
            gb = pl.multiple_of(g * 16, 16)
            a0g = _rne_bf16(a0v[pl.ds(gb, 16)])
            a1g = _rne_bf16(a1v[pl.ds(gb, 16)])
            for i in range(16):
                s0 = a0g[i]
                s1 = a1g[i]
                e = gb + i
                for f in range(NF):
                    sl = pl.ds(f * 16, 16)
                    v = rows[e, sl] + (s0 * w0[f] + s1 * w1[f] + bb[f])
                    rows[e, sl] = jnp.maximum(v, 0.0)

            iot = lax.iota(jnp.int32, 16)
            d16 = dstv[pl.ds(gb, 16)]
            garb = jnp.full((16,), GARB, jnp.int32)
            occ = jnp.zeros((16,), jnp.int32)
            for j in range(16):
                dj = d16[j]
                occ = occ + jnp.where((d16 == dj) & (iot > j), 1, 0)

            om = occ[0]
            for j in range(1, 16):
                om = jnp.maximum(om, occ[j])

            rows16 = rows.at[pl.ds(gb, 16)]
            idx16[...] = jnp.where(occ == 0, d16, garb)
            pltpu.sync_copy(rows16, agg.at[idx16], add=True)

            def wave(k, wcarry):
                idx16[...] = jnp.where(occ == k, d16, garb)
                pltpu.sync_copy(rows16, agg.at[idx16], add=True)
                return wcarry

            lax.fori_loop(1, om + 1, wave, 0)
            return gcarry

        lax.fori_loop(0, CH // 16, group_body, 0)
        return carry

    lax.fori_loop(0, NCHUNK, chunk_body, 0)

    plsc.subcore_barrier()

    def wchunk(k, carry):
        c = sid + k * NS

        @pl.when(c < NZC)
        def _():
            pltpu.sync_copy(agg.at[pl.ds(c * ZB, ZB)],
                            out_hbm.at[cid, pl.ds(c * ZB, ZB)])

        return carry

    lax.fori_loop(0, (NZC + NS - 1) // NS, wchunk, 0)


@jax.jit
def _sc_aggregate(x, src, dst, a0, a1, wb):
    mesh = plsc.VectorSubcoreMesh(core_axis_name="c", subcore_axis_name="s")
    run = pl.kernel(
        _sc_body,
        out_type=jax.ShapeDtypeStruct((NC, N, F), jnp.float32),
        mesh=mesh,
        scratch_types=[
            pltpu.VMEM((CH,), jnp.int32),
            pltpu.VMEM((CH,), jnp.int32),
            pltpu.VMEM((CH,), jnp.float32),
            pltpu.VMEM((CH,), jnp.float32),
            pltpu.VMEM((CH, F), jnp.float32),
            pltpu.VMEM((3, F), jnp.float32),
            pltpu.VMEM((ZB, F), jnp.float32),
            pltpu.VMEM((16,), jnp.int32),
            pltpu.VMEM_SHARED((N + 8, F), jnp.float32),
            pltpu.SemaphoreType.DMA,
        ],
    )
    return run(x, src, dst, a0, a1, wb)


def _bn_leaky(h, g, b):
    m = jnp.mean(h, axis=0, keepdims=True)
    v = jnp.mean((h - m) * (h - m), axis=0, keepdims=True)
    h = (h - m) / jnp.sqrt(v + BN_EPS) * g + b
    return jnp.where(h >= 0.0, h, ALPHA * h)


def _tc_body(eps_ref, x_ref, p_ref, w1_ref, b1_ref, g1_ref, bt1_ref,
             w2_ref, b2_ref, g2_ref, bt2_ref, gn_ref, btn_ref, out_ref):
    h = (1.0 + eps_ref[0, 0]) * x_ref[...] + p_ref[0] + p_ref[1]
    h = jnp.dot(h, w1_ref[...], preferred_element_type=jnp.float32) + b1_ref[...]
    h = _bn_leaky(h, g1_ref[...], bt1_ref[...])
    h = jnp.dot(h, w2_ref[...], preferred_element_type=jnp.float32) + b2_ref[...]
    h = _bn_leaky(h, g2_ref[...], bt2_ref[...])
    out_ref[...] = _bn_leaky(h, gn_ref[...], btn_ref[...])


@jax.jit
def _tc_mlp(eps, x, p, w1, b1, g1, bt1, w2, b2, g2, bt2, gn, btn):
    return pl.pallas_call(
        _tc_body,
        out_shape=jax.ShapeDtypeStruct((N, H), jnp.float32),
    )(eps, x, p, w1, b1, g1, bt1, w2, b2, g2, bt2, gn, btn)


def kernel(x, edge_idx, edge_attr, We, be, W1, b1, g1, bt1, W2, b2, g2,
           bt2, gn, btn, eps):
    src = edge_idx[0]
    dst = edge_idx[1]
    a0 = edge_attr[:, 0]
    a1 = edge_attr[:, 1]
    wb = jnp.concatenate([We, be[None, :]], axis=0)
    partials = _sc_aggregate(x, src, dst, a0, a1, wb)
    eps_arr = jnp.reshape(eps, (1, 1)).astype(jnp.float32)
    return _tc_mlp(eps_arr, x, partials,
                   W1, b1[None, :], g1[None, :], bt1[None, :],
                   W2, b2[None, :], g2[None, :], bt2[None, :],
                   gn[None, :], btn[None, :])


# speedup vs baseline: 2.8368x; 1.1164x over previous
import functools

import jax
import jax.numpy as jnp
from jax import lax
from jax.experimental import pallas as pl
from jax.experimental.pallas import tpu as pltpu
from jax.experimental.pallas import tpu_sc as plsc

N = 10000
E = 320000
F = 128
H = 256
BN_EPS = 1e-5
ALPHA = 0.01

NC = 2
NS = 16
NW = NC * NS
EPW = E // NW
CH = 80
NCHUNK = EPW // CH
ZB = 200
NZC = N // ZB
NF = F // 16


def _rne_bf16(v):
    c = v * jnp.float32(65537.0)
    return c - (c - v)


def _sc_body(x_hbm, src_hbm, dst_hbm, a0_hbm, a1_hbm, wb_hbm, out_hbm,
             srcv, dstv, a0v, a1v, rows, wbv, zbuf, agg, sem):
    cid = lax.axis_index("c")
    sid = lax.axis_index("s")
    wid = sid * NC + cid

    zv = jnp.zeros((16,), jnp.float32)

    def zrow(i, carry):
        for f in range(NF):
            zbuf[i, pl.ds(f * 16, 16)] = zv
        return carry

    lax.fori_loop(0, ZB, zrow, 0)

    def zchunk(k, carry):
        c = sid + k * NS

        @pl.when(c < NZC)
        def _():
            pltpu.sync_copy(zbuf, agg.at[pl.ds(c * ZB, ZB)])

        return carry

    lax.fori_loop(0, (NZC + NS - 1) // NS, zchunk, 0)

    pltpu.sync_copy(wb_hbm, wbv)
    w0 = [_rne_bf16(wbv[0, pl.ds(f * 16, 16)]) for f in range(NF)]
    w1 = [_rne_bf16(wbv[1, pl.ds(f * 16, 16)]) for f in range(NF)]
    bb = [wbv[2, pl.ds(f * 16, 16)] for f in range(NF)]

    plsc.subcore_barrier()

    ebase = wid * EPW

    def chunk_body(c, carry):
        off = ebase + c * CH
        pltpu.sync_copy(src_hbm.at[pl.ds(off, CH)], srcv)
        pltpu.sync_copy(dst_hbm.at[pl.ds(off, CH)], dstv)
        pltpu.sync_copy(a0_hbm.at[pl.ds(off, CH)], a0v)
        pltpu.sync_copy(a1_hbm.at[pl.ds(off, CH)], a1v)
        pltpu.async_copy(x_hbm.at[srcv], rows, sem).wait()

        def group_body(g, gcarry):
            gb = pl.multiple_of(g * 16, 16)
            a0g = _rne_bf16(a0v[pl.ds(gb, 16)])
            a1g = _rne_bf16(a1v[pl.ds(gb, 16)])
            for i in range(16):
                s0 = a0g[i]
                s1 = a1g[i]
                e = gb + i
                for f in range(NF):
                    sl = pl.ds(f * 16, 16)
                    v = rows[e, sl] + (s0 * w0[f] + s1 * w1[f] + bb[f])
                    rows[e, sl] = jnp.maximum(v, 0.0)

            return gcarry

        lax.fori_loop(0, CH // 16, group_body, 0)
        pltpu.sync_copy(rows, agg.at[dstv], add=True)
        return carry

    lax.fori_loop(0, NCHUNK, chunk_body, 0)

    plsc.subcore_barrier()

    def wchunk(k, carry):
        c = sid + k * NS

        @pl.when(c < NZC)
        def _():
            pltpu.sync_copy(agg.at[pl.ds(c * ZB, ZB)],
                            out_hbm.at[cid, pl.ds(c * ZB, ZB)])

        return carry

    lax.fori_loop(0, (NZC + NS - 1) // NS, wchunk, 0)


@jax.jit
def _sc_aggregate(x, src, dst, a0, a1, wb):
    mesh = plsc.VectorSubcoreMesh(core_axis_name="c", subcore_axis_name="s")
    run = pl.kernel(
        _sc_body,
        out_type=jax.ShapeDtypeStruct((NC, N, F), jnp.float32),
        mesh=mesh,
        scratch_types=[
            pltpu.VMEM((CH,), jnp.int32),
            pltpu.VMEM((CH,), jnp.int32),
            pltpu.VMEM((CH,), jnp.float32),
            pltpu.VMEM((CH,), jnp.float32),
            pltpu.VMEM((CH, F), jnp.float32),
            pltpu.VMEM((3, F), jnp.float32),
            pltpu.VMEM((ZB, F), jnp.float32),
            pltpu.VMEM_SHARED((N, F), jnp.float32),
            pltpu.SemaphoreType.DMA,
        ],
    )
    return run(x, src, dst, a0, a1, wb)


def _bn_leaky(h, g, b):
    m = jnp.mean(h, axis=0, keepdims=True)
    v = jnp.mean((h - m) * (h - m), axis=0, keepdims=True)
    h = (h - m) / jnp.sqrt(v + BN_EPS) * g + b
    return jnp.where(h >= 0.0, h, ALPHA * h)


def _tc_body(eps_ref, x_ref, p_ref, w1_ref, b1_ref, g1_ref, bt1_ref,
             w2_ref, b2_ref, g2_ref, bt2_ref, gn_ref, btn_ref, out_ref):
    h = (1.0 + eps_ref[0, 0]) * x_ref[...] + p_ref[0] + p_ref[1]
    h = jnp.dot(h, w1_ref[...], preferred_element_type=jnp.float32) + b1_ref[...]
    h = _bn_leaky(h, g1_ref[...], bt1_ref[...])
    h = jnp.dot(h, w2_ref[...], preferred_element_type=jnp.float32) + b2_ref[...]
    h = _bn_leaky(h, g2_ref[...], bt2_ref[...])
    out_ref[...] = _bn_leaky(h, gn_ref[...], btn_ref[...])


@jax.jit
def _tc_mlp(eps, x, p, w1, b1, g1, bt1, w2, b2, g2, bt2, gn, btn):
    return pl.pallas_call(
        _tc_body,
        out_shape=jax.ShapeDtypeStruct((N, H), jnp.float32),
    )(eps, x, p, w1, b1, g1, bt1, w2, b2, g2, bt2, gn, btn)


def kernel(x, edge_idx, edge_attr, We, be, W1, b1, g1, bt1, W2, b2, g2,
           bt2, gn, btn, eps):
    src = edge_idx[0]
    dst = edge_idx[1]
    a0 = edge_attr[:, 0]
    a1 = edge_attr[:, 1]
    wb = jnp.concatenate([We, be[None, :]], axis=0)
    partials = _sc_aggregate(x, src, dst, a0, a1, wb)
    eps_arr = jnp.reshape(eps, (1, 1)).astype(jnp.float32)
    return _tc_mlp(eps_arr, x, partials,
                   W1, b1[None, :], g1[None, :], bt1[None, :],
                   W2, b2[None, :], g2[None, :], bt2[None, :],
                   gn[None, :], btn[None, :])
